# baseline (speedup 1.0000x reference)
import functools

import jax
import jax.numpy as jnp
from jax import lax
from jax.experimental import pallas as pl
from jax.experimental.pallas import tpu as pltpu
from jax.experimental.pallas import tpu_sc as plsc

N = 10000
F = 128
NPAD = 10112
NC, NS = 2, 16
NW = NC * NS
CHUNK = 64
BLK = 1264
ROWS_PER_SUB = NPAD // NS


NROW = 4
GROUP = 16


def _make_agg(width, chunks_per_worker):
    cpw = chunks_per_worker
    mesh = plsc.VectorSubcoreMesh(
        core_axis_name="c", subcore_axis_name="s",
        num_cores=NC, num_subcores=NS)

    @functools.partial(
        pl.kernel,
        out_type=jax.ShapeDtypeStruct((NC, NPAD, width), jnp.float32),
        mesh=mesh,
        scratch_types=[
            [pltpu.VMEM((CHUNK,), jnp.int32)] * GROUP,
            [pltpu.VMEM((CHUNK,), jnp.int32)] * GROUP,
            [pltpu.VMEM((CHUNK, width), jnp.float32)] * NROW,
            pltpu.VMEM_SHARED((NPAD, width), jnp.float32),
            [pltpu.SemaphoreType.DMA] * NROW,
            [pltpu.SemaphoreType.DMA] * NROW,
            pltpu.SemaphoreType.DMA,
            pltpu.SemaphoreType.DMA,
        ],
    )
    def agg(g_hbm, src_hbm, dst_hbm, zeros_hbm, out_hbm,
            sib, dib, rows, acc, gsems, ssems, ssem, dsem):
        c = lax.axis_index("c")
        s = lax.axis_index("s")
        w = s * NC + c
        r0 = s * ROWS_PER_SUB
        base = w * cpw * CHUNK
        pltpu.sync_copy(zeros_hbm.at[pl.ds(r0, ROWS_PER_SUB)],
                        acc.at[pl.ds(r0, ROWS_PER_SUB)])
        plsc.subcore_barrier()

        def body(gi, carry):
            off = base + gi * (GROUP * CHUNK)
            sls = [pltpu.async_copy(
                src_hbm.at[pl.ds(off + k * CHUNK, CHUNK)], sib[k], ssem)
                for k in range(GROUP)]
            dds = [pltpu.async_copy(
                dst_hbm.at[pl.ds(off + k * CHUNK, CHUNK)], dib[k], dsem)
                for k in range(GROUP)]
            for d in sls:
                d.wait()
            for d in dds:
                d.wait()

            def gather(k):
                return pltpu.async_copy(
                    g_hbm.at[sib[k]], rows[k % NROW], gsems[k % NROW])

            def scatter(k):
                return pltpu.async_copy(
                    rows[k % NROW], acc.at[dib[k]],
                    ssems[k % NROW], add=True)

            gds = [None] * GROUP
            sds = [None] * GROUP
            gds[0] = gather(0)
            gds[1] = gather(1)
            for t in range(GROUP):
                if t >= 2:
                    sds[t - 2].wait()
                if t + 2 < GROUP:
                    gds[t + 2] = gather(t + 2)
                gds[t].wait()
                sds[t] = scatter(t)
            sds[GROUP - 2].wait()
            sds[GROUP - 1].wait()
            return carry

        lax.fori_loop(0, cpw // GROUP, body, 0)
        plsc.subcore_barrier()
        pltpu.sync_copy(acc.at[pl.ds(r0, ROWS_PER_SUB)],
                        out_hbm.at[c].at[pl.ds(r0, ROWS_PER_SUB)])

    return agg


def _row_specs(*widths):
    return [pl.BlockSpec((BLK, w), lambda i: (i, 0)) for w in widths]


def _full_spec(shape):
    return pl.BlockSpec(shape, lambda i: (0, 0))


def _prologue_body(d0_ref, d1_ref, x_ref, win_ref, bin_ref, w1_ref,
                   dinv_ref, g1_ref):
    i = pl.program_id(0)
    deg = d0_ref[...][:, 0:1] + d1_ref[...][:, 0:1] + 1.0
    dinv = lax.rsqrt(deg)
    row = lax.broadcasted_iota(jnp.int32, (BLK, 1), 0) + i * BLK
    dinv = jnp.where(row < N, dinv, 0.0)
    dinv_ref[...] = dinv
    h1 = x_ref[...] * win_ref[...] + bin_ref[...]
    g1_ref[...] = jnp.dot(h1, w1_ref[...],
                          preferred_element_type=jnp.float32) * dinv


_prologue = pl.pallas_call(
    _prologue_body,
    grid=(NPAD // BLK,),
    in_specs=_row_specs(F, F, 1) + [_full_spec((1, F)), _full_spec((1, F)),
                                      _full_spec((F, F))],
    out_specs=_row_specs(1, F),
    out_shape=(jax.ShapeDtypeStruct((NPAD, 1), jnp.float32),
               jax.ShapeDtypeStruct((NPAD, F), jnp.float32)),
)


def _step_body(p0_ref, p1_ref, g_ref, dinv_ref, b_ref, w_ref, out_ref):
    dinv = dinv_ref[...]
    h = (p0_ref[...] + p1_ref[...] + g_ref[...]) * dinv + b_ref[...]
    h = jnp.maximum(h, 0.0)
    out_ref[...] = jnp.dot(h, w_ref[...],
                           preferred_element_type=jnp.float32) * dinv


_step = pl.pallas_call(
    _step_body,
    grid=(NPAD // BLK,),
    in_specs=_row_specs(F, F, F, 1) + [_full_spec((1, F)), _full_spec((F, F))],
    out_specs=_row_specs(F)[0],
    out_shape=jax.ShapeDtypeStruct((NPAD, F), jnp.float32),
)


def _final_body(p0_ref, p1_ref, g_ref, dinv_ref, b_ref, wo1_ref, bo1_ref,
                wo2_ref, bo2_ref, out_ref):
    h = (p0_ref[...] + p1_ref[...] + g_ref[...]) * dinv_ref[...] + b_ref[...]
    h = jnp.maximum(h, 0.0)
    t = jnp.dot(h, wo1_ref[...], preferred_element_type=jnp.float32)
    t = jnp.maximum(t + bo1_ref[...], 0.0)
    out_ref[...] = jnp.dot(t, wo2_ref[...],
                           preferred_element_type=jnp.float32) + bo2_ref[...]


_final = pl.pallas_call(
    _final_body,
    grid=(NPAD // BLK,),
    in_specs=_row_specs(F, F, F, 1) + [
        _full_spec((1, F)), _full_spec((F, 1024)), _full_spec((1, 1024)),
        _full_spec((1024, 1)), _full_spec((1, 1))],
    out_specs=_row_specs(1)[0],
    out_shape=jax.ShapeDtypeStruct((NPAD, 1), jnp.float32),
)


def kernel(x, edge_index, fc_in_W, fc_in_b, conv1_W, conv1_b, conv2_W,
           conv2_b, conv3_W, conv3_b, conv4_W, conv4_b, fc_out1_W, fc_out1_b,
           fc_out2_W, fc_out2_b):
    e = edge_index.shape[1]
    chunks_per_worker = -(-e // (NW * CHUNK))
    chunks_per_worker = -(-chunks_per_worker // GROUP) * GROUP
    epad = NW * CHUNK * chunks_per_worker
    pad = N + jnp.arange(epad - e, dtype=jnp.int32) % (NPAD - N)
    src = jnp.concatenate([edge_index[0].astype(jnp.int32), pad])
    dst = jnp.concatenate([edge_index[1].astype(jnp.int32), pad])

    agg_f = _make_agg(F, chunks_per_worker)

    zeros128 = jnp.zeros((NPAD, F), jnp.float32)
    ones128 = jnp.ones((NPAD, F), jnp.float32)

    dpart = agg_f(ones128, src, dst, zeros128)

    x_pad = jnp.zeros((NPAD, 1), jnp.float32).at[:N].set(x)
    dinv, g = _prologue(dpart[0], dpart[1], x_pad, fc_in_W,
                        fc_in_b.reshape(1, F), conv1_W)

    convs = [(conv1_W, conv1_b), (conv2_W, conv2_b), (conv3_W, conv3_b),
             (conv4_W, conv4_b)]
    seq = convs * 4
    for li in range(len(seq) - 1):
        _, b_prev = seq[li]
        w_next, _ = seq[li + 1]
        p = agg_f(g, src, dst, zeros128)
        g = _step(p[0], p[1], g, dinv, b_prev.reshape(1, F), w_next)

    p = agg_f(g, src, dst, zeros128)
    out = _final(p[0], p[1], g, dinv, conv4_b.reshape(1, F), fc_out1_W,
                 fc_out1_b.reshape(1, 1024), fc_out2_W,
                 fc_out2_b.reshape(1, 1))
    return out[:N]


# speedup vs baseline: 1.0289x; 1.0289x over previous
import functools

import jax
import jax.numpy as jnp
from jax import lax
from jax.experimental import pallas as pl
from jax.experimental.pallas import tpu as pltpu
from jax.experimental.pallas import tpu_sc as plsc

N = 10000
F = 128
NPAD = 10112
NC, NS = 2, 16
NW = NC * NS
CHUNK = 128
BLK = 1264
ROWS_PER_SUB = NPAD // NS


NROW = 2
GROUP = 16


def _make_agg(width, chunks_per_worker):
    cpw = chunks_per_worker
    mesh = plsc.VectorSubcoreMesh(
        core_axis_name="c", subcore_axis_name="s",
        num_cores=NC, num_subcores=NS)

    @functools.partial(
        pl.kernel,
        out_type=jax.ShapeDtypeStruct((NC, NPAD, width), jnp.float32),
        mesh=mesh,
        scratch_types=[
            [pltpu.VMEM((CHUNK,), jnp.int32)] * GROUP,
            [pltpu.VMEM((CHUNK,), jnp.int32)] * GROUP,
            [pltpu.VMEM((CHUNK, width), jnp.float32)] * NROW,
            pltpu.VMEM_SHARED((NPAD, width), jnp.float32),
            [pltpu.SemaphoreType.DMA] * NROW,
            [pltpu.SemaphoreType.DMA] * NROW,
            pltpu.SemaphoreType.DMA,
            pltpu.SemaphoreType.DMA,
        ],
    )
    def agg(g_hbm, src_hbm, dst_hbm, zeros_hbm, out_hbm,
            sib, dib, rows, acc, gsems, ssems, ssem, dsem):
        c = lax.axis_index("c")
        s = lax.axis_index("s")
        w = s * NC + c
        r0 = s * ROWS_PER_SUB
        base = w * cpw * CHUNK
        pltpu.sync_copy(zeros_hbm.at[pl.ds(r0, ROWS_PER_SUB)],
                        acc.at[pl.ds(r0, ROWS_PER_SUB)])
        plsc.subcore_barrier()

        def body(gi, carry):
            off = base + gi * (GROUP * CHUNK)
            sls = [pltpu.async_copy(
                src_hbm.at[pl.ds(off + k * CHUNK, CHUNK)], sib[k], ssem)
                for k in range(GROUP)]
            dds = [pltpu.async_copy(
                dst_hbm.at[pl.ds(off + k * CHUNK, CHUNK)], dib[k], dsem)
                for k in range(GROUP)]
            for d in sls:
                d.wait()
            for d in dds:
                d.wait()

            def gather(k):
                return pltpu.async_copy(
                    g_hbm.at[sib[k]], rows[k % NROW], gsems[k % NROW])

            def scatter(k):
                return pltpu.async_copy(
                    rows[k % NROW], acc.at[dib[k]],
                    ssems[k % NROW], add=True)

            gds = [None] * GROUP
            sds = [None] * GROUP
            gds[0] = gather(0)
            gds[1] = gather(1)
            for t in range(GROUP + 1):
                if t >= 2:
                    sds[t - 2].wait()
                    if t < GROUP:
                        gds[t] = gather(t)
                if t >= 1:
                    gds[t - 1].wait()
                    sds[t - 1] = scatter(t - 1)
            sds[GROUP - 1].wait()
            return carry

        lax.fori_loop(0, cpw // GROUP, body, 0)
        plsc.subcore_barrier()
        pltpu.sync_copy(acc.at[pl.ds(r0, ROWS_PER_SUB)],
                        out_hbm.at[c].at[pl.ds(r0, ROWS_PER_SUB)])

    return agg


def _row_specs(*widths):
    return [pl.BlockSpec((BLK, w), lambda i: (i, 0)) for w in widths]


def _full_spec(shape):
    return pl.BlockSpec(shape, lambda i: (0, 0))


def _prologue_body(d0_ref, d1_ref, x_ref, win_ref, bin_ref, w1_ref,
                   dinv_ref, g1_ref):
    i = pl.program_id(0)
    deg = d0_ref[...][:, 0:1] + d1_ref[...][:, 0:1] + 1.0
    dinv = lax.rsqrt(deg)
    row = lax.broadcasted_iota(jnp.int32, (BLK, 1), 0) + i * BLK
    dinv = jnp.where(row < N, dinv, 0.0)
    dinv_ref[...] = dinv
    h1 = x_ref[...] * win_ref[...] + bin_ref[...]
    g1_ref[...] = jnp.dot(h1, w1_ref[...],
                          preferred_element_type=jnp.float32) * dinv


_prologue = pl.pallas_call(
    _prologue_body,
    grid=(NPAD // BLK,),
    in_specs=_row_specs(F, F, 1) + [_full_spec((1, F)), _full_spec((1, F)),
                                      _full_spec((F, F))],
    out_specs=_row_specs(1, F),
    out_shape=(jax.ShapeDtypeStruct((NPAD, 1), jnp.float32),
               jax.ShapeDtypeStruct((NPAD, F), jnp.float32)),
)


def _step_body(p0_ref, p1_ref, g_ref, dinv_ref, b_ref, w_ref, out_ref):
    dinv = dinv_ref[...]
    h = (p0_ref[...] + p1_ref[...] + g_ref[...]) * dinv + b_ref[...]
    h = jnp.maximum(h, 0.0)
    out_ref[...] = jnp.dot(h, w_ref[...],
                           preferred_element_type=jnp.float32) * dinv


_step = pl.pallas_call(
    _step_body,
    grid=(NPAD // BLK,),
    in_specs=_row_specs(F, F, F, 1) + [_full_spec((1, F)), _full_spec((F, F))],
    out_specs=_row_specs(F)[0],
    out_shape=jax.ShapeDtypeStruct((NPAD, F), jnp.float32),
)


def _final_body(p0_ref, p1_ref, g_ref, dinv_ref, b_ref, wo1_ref, bo1_ref,
                wo2_ref, bo2_ref, out_ref):
    h = (p0_ref[...] + p1_ref[...] + g_ref[...]) * dinv_ref[...] + b_ref[...]
    h = jnp.maximum(h, 0.0)
    t = jnp.dot(h, wo1_ref[...], preferred_element_type=jnp.float32)
    t = jnp.maximum(t + bo1_ref[...], 0.0)
    out_ref[...] = jnp.dot(t, wo2_ref[...],
                           preferred_element_type=jnp.float32) + bo2_ref[...]


_final = pl.pallas_call(
    _final_body,
    grid=(NPAD // BLK,),
    in_specs=_row_specs(F, F, F, 1) + [
        _full_spec((1, F)), _full_spec((F, 1024)), _full_spec((1, 1024)),
        _full_spec((1024, 1)), _full_spec((1, 1))],
    out_specs=_row_specs(1)[0],
    out_shape=jax.ShapeDtypeStruct((NPAD, 1), jnp.float32),
)


def kernel(x, edge_index, fc_in_W, fc_in_b, conv1_W, conv1_b, conv2_W,
           conv2_b, conv3_W, conv3_b, conv4_W, conv4_b, fc_out1_W, fc_out1_b,
           fc_out2_W, fc_out2_b):
    e = edge_index.shape[1]
    chunks_per_worker = -(-e // (NW * CHUNK))
    chunks_per_worker = -(-chunks_per_worker // GROUP) * GROUP
    epad = NW * CHUNK * chunks_per_worker
    pad = N + jnp.arange(epad - e, dtype=jnp.int32) % (NPAD - N)
    src = jnp.concatenate([edge_index[0].astype(jnp.int32), pad])
    dst = jnp.concatenate([edge_index[1].astype(jnp.int32), pad])

    agg_f = _make_agg(F, chunks_per_worker)

    zeros128 = jnp.zeros((NPAD, F), jnp.float32)
    ones128 = jnp.ones((NPAD, F), jnp.float32)

    dpart = agg_f(ones128, src, dst, zeros128)

    x_pad = jnp.zeros((NPAD, 1), jnp.float32).at[:N].set(x)
    dinv, g = _prologue(dpart[0], dpart[1], x_pad, fc_in_W,
                        fc_in_b.reshape(1, F), conv1_W)

    convs = [(conv1_W, conv1_b), (conv2_W, conv2_b), (conv3_W, conv3_b),
             (conv4_W, conv4_b)]
    seq = convs * 4
    for li in range(len(seq) - 1):
        _, b_prev = seq[li]
        w_next, _ = seq[li + 1]
        p = agg_f(g, src, dst, zeros128)
        g = _step(p[0], p[1], g, dinv, b_prev.reshape(1, F), w_next)

    p = agg_f(g, src, dst, zeros128)
    out = _final(p[0], p[1], g, dinv, conv4_b.reshape(1, F), fc_out1_W,
                 fc_out1_b.reshape(1, 1024), fc_out2_W,
                 fc_out2_b.reshape(1, 1))
    return out[:N]
